# baseline (speedup 1.0000x reference)
import functools

import jax
import jax.numpy as jnp
from jax.experimental import pallas as pl
from jax.experimental.pallas import tpu as pltpu

N = 8192
D = 512
BM = 128
NSTRIPS = N // BM


def _compute_dist(xr_ref, xf_ref, x2r_ref, x2c_ref, dst_ref):
    g = jax.lax.dot_general(
        xr_ref[...], xf_ref[...],
        dimension_numbers=(((1,), (1,)), ((), ())),
        preferred_element_type=jnp.float32,
    )
    dst_ref[...] = (x2r_ref[...] + x2c_ref[...]) - 2.0 * g


def _select_writeout(src_ref, out_ref):
    dist = src_ref[...]
    jidx = jax.lax.broadcasted_iota(jnp.int32, (BM, N), 1)
    i1 = jnp.argmin(dist, axis=1, keepdims=True).astype(jnp.int32)
    d2 = jnp.where(jidx == i1, jnp.inf, dist)
    i2 = jnp.argmin(d2, axis=1, keepdims=True).astype(jnp.int32)
    out_ref[...] = ((jidx == i1) | (jidx == i2)).astype(jnp.float32)


def _knn_adj_kernel(xr_ref, xf_ref, x2r_ref, x2c_ref, out_ref, da_ref, db_ref):
    i = pl.program_id(0)
    even = jax.lax.rem(i, 2) == 0

    @pl.when(even)
    def _():
        _compute_dist(xr_ref, xf_ref, x2r_ref, x2c_ref, da_ref)
        _select_writeout(db_ref, out_ref)

    @pl.when(jnp.logical_not(even))
    def _():
        _compute_dist(xr_ref, xf_ref, x2r_ref, x2c_ref, db_ref)
        _select_writeout(da_ref, out_ref)


@functools.partial(jax.jit, static_argnames=("interpret",))
def kernel(X, interpret=False):
    x2 = jnp.sum(X * X, axis=1)
    x2_col = x2.reshape(N, 1)
    x2_row = x2.reshape(1, N)
    return pl.pallas_call(
        _knn_adj_kernel,
        grid=(NSTRIPS + 1,),
        in_specs=[
            pl.BlockSpec((BM, D), lambda i: (jnp.minimum(i, NSTRIPS - 1), 0)),
            pl.BlockSpec((N, D), lambda i: (0, 0)),
            pl.BlockSpec((BM, 1), lambda i: (jnp.minimum(i, NSTRIPS - 1), 0)),
            pl.BlockSpec((1, N), lambda i: (0, 0)),
        ],
        out_specs=pl.BlockSpec((BM, N), lambda i: (jnp.maximum(i - 1, 0), 0)),
        out_shape=jax.ShapeDtypeStruct((N, N), jnp.float32),
        scratch_shapes=[
            pltpu.VMEM((BM, N), jnp.float32),
            pltpu.VMEM((BM, N), jnp.float32),
        ],
        interpret=interpret,
    )(X, X, x2_col, x2_row)


# speedup vs baseline: 1.6429x; 1.6429x over previous
import functools

import jax
import jax.numpy as jnp
from jax.experimental import pallas as pl
from jax.experimental.pallas import tpu as pltpu

N = 8192
D = 512
BM = 256


def _knn_adj_kernel(xr_ref, xf_ref, x2r_ref, x2c_ref, out_ref):
    rows = xr_ref[...]
    g = jax.lax.dot_general(
        rows, xf_ref[...],
        dimension_numbers=(((1,), (1,)), ((), ())),
        preferred_element_type=jnp.float32,
    )
    dist = (x2r_ref[...] + x2c_ref[...]) - 2.0 * g
    jidx = jax.lax.broadcasted_iota(jnp.int32, (BM, N), 1)
    i1 = jnp.argmin(dist, axis=1, keepdims=True).astype(jnp.int32)
    d2 = jnp.where(jidx == i1, jnp.inf, dist)
    i2 = jnp.argmin(d2, axis=1, keepdims=True).astype(jnp.int32)
    out_ref[...] = ((jidx == i1) | (jidx == i2)).astype(jnp.float32)


@functools.partial(jax.jit, static_argnames=("interpret",))
def kernel(X, interpret=False):
    x2 = jnp.sum(X * X, axis=1)
    x2_col = x2.reshape(N, 1)
    x2_row = x2.reshape(1, N)
    return pl.pallas_call(
        _knn_adj_kernel,
        grid=(N // BM,),
        in_specs=[
            pl.BlockSpec((BM, D), lambda i: (i, 0)),
            pl.BlockSpec((N, D), lambda i: (0, 0)),
            pl.BlockSpec((BM, 1), lambda i: (i, 0)),
            pl.BlockSpec((1, N), lambda i: (0, 0)),
        ],
        out_specs=pl.BlockSpec((BM, N), lambda i: (i, 0)),
        out_shape=jax.ShapeDtypeStruct((N, N), jnp.float32),
        interpret=interpret,
    )(X, X, x2_col, x2_row)


# speedup vs baseline: 1.6442x; 1.0008x over previous
import functools

import jax
import jax.numpy as jnp
from jax.experimental import pallas as pl
from jax.experimental.pallas import tpu as pltpu

N = 8192
D = 512
BM = 256


def _knn_adj_kernel(xr_ref, xf_ref, x2r_ref, x2c_ref, out_ref):
    rows = xr_ref[...]
    g = jax.lax.dot_general(
        rows, xf_ref[...],
        dimension_numbers=(((1,), (1,)), ((), ())),
        preferred_element_type=jnp.float32,
    )
    dist = (x2r_ref[...] + x2c_ref[...]) - 2.0 * g
    jidx = jax.lax.broadcasted_iota(jnp.int32, (BM, N), 1)
    i1 = jnp.argmin(dist, axis=1, keepdims=True).astype(jnp.int32)
    d2 = jnp.where(jidx == i1, jnp.inf, dist)
    i2 = jnp.argmin(d2, axis=1, keepdims=True).astype(jnp.int32)
    out_ref[...] = ((jidx == i1) | (jidx == i2)).astype(jnp.float32)


@functools.partial(jax.jit, static_argnames=("interpret",))
def kernel(X, interpret=False):
    x2 = jnp.sum(X * X, axis=1)
    x2_col = x2.reshape(N, 1)
    x2_row = x2.reshape(1, N)
    return pl.pallas_call(
        _knn_adj_kernel,
        grid=(N // BM,),
        in_specs=[
            pl.BlockSpec((BM, D), lambda i: (i, 0)),
            pl.BlockSpec((N, D), lambda i: (0, 0)),
            pl.BlockSpec((BM, 1), lambda i: (i, 0)),
            pl.BlockSpec((1, N), lambda i: (0, 0)),
        ],
        out_specs=pl.BlockSpec((BM, N), lambda i: (i, 0)),
        out_shape=jax.ShapeDtypeStruct((N, N), jnp.float32),
        compiler_params=pltpu.CompilerParams(
            dimension_semantics=("parallel",),
        ),
        interpret=interpret,
    )(X, X, x2_col, x2_row)


# speedup vs baseline: 1.6467x; 1.0015x over previous
import functools

import jax
import jax.numpy as jnp
from jax.experimental import pallas as pl
from jax.experimental.pallas import tpu as pltpu

N = 8192
D = 512
BM = 256


def _knn_adj_kernel(xr_ref, xf_ref, x2r_ref, x2c_ref, out_ref):
    rows = xr_ref[...]
    g = jax.lax.dot_general(
        rows, xf_ref[...],
        dimension_numbers=(((1,), (1,)), ((), ())),
        preferred_element_type=jnp.float32,
    )
    dist = (x2r_ref[...] + x2c_ref[...]) - 2.0 * g
    jidx = jax.lax.broadcasted_iota(jnp.int32, (BM, N), 1)
    i1 = jnp.argmin(dist, axis=1, keepdims=True).astype(jnp.int32)
    d2 = jnp.where(jidx == i1, jnp.inf, dist)
    i2 = jnp.argmin(d2, axis=1, keepdims=True).astype(jnp.int32)
    out_ref[...] = ((jidx == i1) | (jidx == i2)).astype(jnp.float32)


@functools.partial(jax.jit, static_argnames=("interpret",))
def kernel(X, interpret=False):
    x2 = jnp.sum(X * X, axis=1)
    x2_col = x2.reshape(N, 1)
    x2_row = x2.reshape(1, N)
    return pl.pallas_call(
        _knn_adj_kernel,
        grid=(N // BM,),
        in_specs=[
            pl.BlockSpec((BM, D), lambda i: (i, 0)),
            pl.BlockSpec((N, D), lambda i: (0, 0)),
            pl.BlockSpec((BM, 1), lambda i: (i, 0)),
            pl.BlockSpec((1, N), lambda i: (0, 0)),
        ],
        out_specs=pl.BlockSpec((BM, N), lambda i: (i, 0)),
        out_shape=jax.ShapeDtypeStruct((N, N), jnp.float32),
        interpret=interpret,
    )(X, X, x2_col, x2_row)
